# baseline (speedup 1.0000x reference)
import functools

import jax
import jax.numpy as jnp
from jax.experimental import pallas as pl
from jax.experimental.pallas import tpu as pltpu

_VMEM_LIMIT = 38 * 1024 * 1024
_QSCALE = 16000.0


def _symlog(x):
    return jnp.sign(x) * jnp.log1p(jnp.abs(x))


def _sum_hw(x):
    return jnp.sum(jnp.sum(x, axis=2), axis=1, keepdims=True)


def _coords(row_block, th, w, inv_r, inv_c):
    row0 = row_block * th
    rows = (jax.lax.broadcasted_iota(jnp.int32, (th, w), 0) + row0
            ).astype(jnp.float32) * inv_r
    cols = jax.lax.broadcasted_iota(jnp.int32, (th, w), 1).astype(jnp.float32) * inv_c
    return rows, cols


def _suma_kernel(mask_ref, sig_ref, out_ref, *, th, w, inv_r, inv_c):
    mask = mask_ref[0].astype(jnp.float32)
    sig = sig_ref[0, 0].astype(jnp.float32)
    rows, cols = _coords(pl.program_id(1), th, w, inv_r, inv_c)
    msig = mask * sig[None]
    out = jnp.concatenate([
        _sum_hw(mask),
        _sum_hw(mask * rows[None]),
        _sum_hw(mask * cols[None]),
        _sum_hw(msig),
        _sum_hw(msig * msig),
    ], axis=0)
    out_ref[...] = out[None, None]


def _emitb_kernel(mask_ref, emb_ref, seed_ref, par_ref, key_ref, stats_ref, *,
                  th, w, inv_r, inv_c):
    mask = mask_ref[0].astype(jnp.float32)
    rows, cols = _coords(pl.program_id(1), th, w, inv_r, inv_c)
    e0 = _symlog(emb_ref[0, 0].astype(jnp.float32)) + rows
    e1 = _symlog(emb_ref[0, 1].astype(jnp.float32)) + cols
    seed = jax.nn.sigmoid(seed_ref[0, 0].astype(jnp.float32))

    par = par_ref[0]
    c0 = par[:, :, 0:1]
    c1 = par[:, :, 1:2]
    sw = par[:, :, 2:3]

    d0 = e0[None] - c0
    d1 = e1[None] - c1
    dist = jnp.exp(-(sw * (d0 * d0 + d1 * d1)))
    logit = (2.0 * dist - 1.0).astype(jnp.bfloat16).astype(jnp.float32)
    sign = 2.0 * mask - 1.0
    err = 1.0 - logit * sign
    q = jnp.round(err * _QSCALE).astype(jnp.int32)
    lab = mask.astype(jnp.int32)
    b = pl.program_id(0)
    mi = jax.lax.broadcasted_iota(jnp.int32, q.shape, 0)
    off = (b * q.shape[0] + mi) * 65536
    key_ref[...] = (off + q * 2 + lab)[None]

    st = (seed[None] - dist) * mask
    stats_ref[...] = _sum_hw(st * st)[None, None]


def _pick_th(h):
    for th in (128, 64, 32, 16, 8):
        if h % th == 0:
            return th
    return h


def kernel(features, masks):
    B, F, H, W = features.shape
    _, M, _, _ = masks.shape
    HW = H * W
    img_size = (1024, 2048)
    inv_r = 1.0 / (img_size[0] - 1)
    inv_c = 1.0 / (img_size[1] - 1)
    TH = _pick_th(H)
    nT = H // TH
    w_inst, w_var, w_seed = 1.0, 0.01, 0.01

    partials = pl.pallas_call(
        functools.partial(_suma_kernel, th=TH, w=W, inv_r=inv_r, inv_c=inv_c),
        out_shape=jax.ShapeDtypeStruct((B, nT, 5 * M, 1), jnp.float32),
        grid=(B, nT),
        in_specs=[pl.BlockSpec((1, M, TH, W), lambda b, t: (b, 0, t, 0)),
                  pl.BlockSpec((1, 1, TH, W), lambda b, t: (b, 2, t, 0))],
        out_specs=pl.BlockSpec((1, 1, 5 * M, 1), lambda b, t: (b, t, 0, 0)),
        compiler_params=pltpu.CompilerParams(
            dimension_semantics=("parallel", "parallel"),
            vmem_limit_bytes=_VMEM_LIMIT),
    )(masks, features)

    sums = jnp.sum(partials[..., 0], axis=1).reshape(B, 5, M)
    msum = sums[:, 0]
    denom = msum + 1e-6
    crow = sums[:, 1] / denom
    ccol = sums[:, 2] / denom
    ssum = sums[:, 3]
    s2sum = sums[:, 4]
    s = ssum / denom
    var_loss = jnp.sum(s2sum - 2.0 * s * ssum + HW * s * s) / (B * M * HW)
    s_w = jnp.exp(10.0 * s)
    params = jnp.stack([crow, ccol, s_w], axis=-1)[:, :, None, :]

    keys, seed_parts = pl.pallas_call(
        functools.partial(_emitb_kernel, th=TH, w=W, inv_r=inv_r, inv_c=inv_c),
        out_shape=(jax.ShapeDtypeStruct((B, M, H, W), jnp.int32),
                   jax.ShapeDtypeStruct((B, nT, M, 1), jnp.float32)),
        grid=(B, nT),
        in_specs=[pl.BlockSpec((1, M, TH, W), lambda b, t: (b, 0, t, 0)),
                  pl.BlockSpec((1, 2, TH, W), lambda b, t: (b, 0, t, 0)),
                  pl.BlockSpec((1, 1, TH, W), lambda b, t: (b, 3, t, 0)),
                  pl.BlockSpec((1, M, 1, 3), lambda b, t: (b, 0, 0, 0))],
        out_specs=[pl.BlockSpec((1, M, TH, W), lambda b, t: (b, 0, t, 0)),
                   pl.BlockSpec((1, 1, M, 1), lambda b, t: (b, t, 0, 0))],
        compiler_params=pltpu.CompilerParams(
            dimension_semantics=("parallel", "parallel"),
            vmem_limit_bytes=_VMEM_LIMIT),
    )(masks, features, features, params)

    seed_loss = jnp.sum(seed_parts) / (B * M * HW)

    flat = keys.reshape(B * M, HW)
    hist = jnp.zeros((B * M * 65536,), jnp.float32).at[flat.reshape(-1)].add(1.0)
    hist = hist.reshape(B * M, 32768, 2)
    hneg, hpos = hist[..., 0], hist[..., 1]
    cnt = hneg + hpos
    tot = jnp.sum(cnt, axis=1, keepdims=True)
    ptot = jnp.sum(hpos, axis=1, keepdims=True)
    i1 = tot - jnp.cumsum(cnt, axis=1) + cnt
    k1 = ptot - jnp.cumsum(hpos, axis=1) + hpos
    i0 = i1 - cnt
    k0 = k1 - hpos
    e_v = jnp.arange(32768, dtype=jnp.float32)[None, :] * (1.0 / _QSCALE)

    def _jac(i, k):
        den = ptot + i - k
        return 1.0 - (ptot - k) / jnp.where(den == 0, 1.0, den)

    inst_loss = jnp.mean(jnp.sum(e_v * (_jac(i1, k1) - _jac(i0, k0)), axis=1))

    return w_inst * inst_loss + w_var * var_loss + w_seed * seed_loss


# speedup vs baseline: 4.7843x; 4.7843x over previous
import functools

import jax
import jax.numpy as jnp
from jax.experimental import pallas as pl
from jax.experimental.pallas import tpu as pltpu

_VMEM_LIMIT = 38 * 1024 * 1024
_QSCALE = 16000.0


def _symlog(x):
    return jnp.sign(x) * jnp.log1p(jnp.abs(x))


def _sum_hw(x):
    return jnp.sum(jnp.sum(x, axis=2), axis=1, keepdims=True)


def _coords(row_block, th, w, inv_r, inv_c):
    row0 = row_block * th
    rows = (jax.lax.broadcasted_iota(jnp.int32, (th, w), 0) + row0
            ).astype(jnp.float32) * inv_r
    cols = jax.lax.broadcasted_iota(jnp.int32, (th, w), 1).astype(jnp.float32) * inv_c
    return rows, cols


def _suma_kernel(mask_ref, sig_ref, out_ref, *, th, w, inv_r, inv_c):
    mask = mask_ref[0].astype(jnp.float32)
    sig = sig_ref[0, 0].astype(jnp.float32)
    rows, cols = _coords(pl.program_id(1), th, w, inv_r, inv_c)
    msig = mask * sig[None]
    out = jnp.concatenate([
        _sum_hw(mask),
        _sum_hw(mask * rows[None]),
        _sum_hw(mask * cols[None]),
        _sum_hw(msig),
        _sum_hw(msig * msig),
    ], axis=0)
    out_ref[...] = out[None, None]


def _emitb_kernel(mask_ref, emb_ref, seed_ref, par_ref, key_ref, stats_ref, *,
                  th, w, inv_r, inv_c):
    mask = mask_ref[0].astype(jnp.float32)
    rows, cols = _coords(pl.program_id(1), th, w, inv_r, inv_c)
    e0 = _symlog(emb_ref[0, 0].astype(jnp.float32)) + rows
    e1 = _symlog(emb_ref[0, 1].astype(jnp.float32)) + cols
    seed = jax.nn.sigmoid(seed_ref[0, 0].astype(jnp.float32))

    par = par_ref[0]
    c0 = par[:, :, 0:1]
    c1 = par[:, :, 1:2]
    sw = par[:, :, 2:3]

    d0 = e0[None] - c0
    d1 = e1[None] - c1
    dist = jnp.exp(-(sw * (d0 * d0 + d1 * d1)))
    logit = (2.0 * dist - 1.0).astype(jnp.bfloat16).astype(jnp.float32)
    sign = 2.0 * mask - 1.0
    err = 1.0 - logit * sign
    q = jnp.round(err * _QSCALE).astype(jnp.int32)
    lab = mask.astype(jnp.int32)
    key_ref[...] = (q * 2 + lab).astype(jnp.uint16)[None]

    st = (seed[None] - dist) * mask
    stats_ref[...] = _sum_hw(st * st)[None, None]


def _tail_kernel(sk_ref, out_ref, *, n, qinv):
    k32 = sk_ref[0].astype(jnp.int32)
    s_dim = k32.shape[0]
    g = jnp.bitwise_and(k32, 1).astype(jnp.float32)
    e = jnp.right_shift(k32, 1).astype(jnp.float32) * qinv
    ur = jax.lax.broadcasted_iota(jnp.int32, (128, 128), 0)
    uc = jax.lax.broadcasted_iota(jnp.int32, (128, 128), 1)
    tri = jnp.where(ur >= uc, 1.0, 0.0).astype(jnp.bfloat16)
    kl = jnp.dot(g.astype(jnp.bfloat16), tri,
                 preferred_element_type=jnp.float32)
    rowtot = jnp.broadcast_to(kl[:, 0:1], kl.shape)
    def _shift_up(x, d):
        return jnp.concatenate(
            [x[d:], jnp.zeros((d, 128), jnp.float32)], axis=0)
    t = _shift_up(rowtot, 1)
    d = 1
    while d < s_dim:
        t = t + _shift_up(t, d)
        d *= 2
    kk = kl + t
    p_tot = kk[0:1, 0:1]
    f = (jax.lax.broadcasted_iota(jnp.int32, kk.shape, 0) * 128
         + jax.lax.broadcasted_iota(jnp.int32, kk.shape, 1)).astype(jnp.float32)
    p = n - f

    def _jac(i, k):
        den = p_tot + i - k
        return 1.0 - (p_tot - k) / jnp.where(den == 0.0, 1.0, den)

    val = e * (_jac(p, kk) - _jac(p - 1.0, kk - g))
    out_ref[...] = jnp.sum(jnp.sum(val, axis=1), axis=0, keepdims=True)[None, None]


def _pick_th(h):
    for th in (128, 64, 32, 16, 8):
        if h % th == 0:
            return th
    return h


def kernel(features, masks):
    B, F, H, W = features.shape
    _, M, _, _ = masks.shape
    HW = H * W
    img_size = (1024, 2048)
    inv_r = 1.0 / (img_size[0] - 1)
    inv_c = 1.0 / (img_size[1] - 1)
    TH = _pick_th(H)
    nT = H // TH
    w_inst, w_var, w_seed = 1.0, 0.01, 0.01

    partials = pl.pallas_call(
        functools.partial(_suma_kernel, th=TH, w=W, inv_r=inv_r, inv_c=inv_c),
        out_shape=jax.ShapeDtypeStruct((B, nT, 5 * M, 1), jnp.float32),
        grid=(B, nT),
        in_specs=[pl.BlockSpec((1, M, TH, W), lambda b, t: (b, 0, t, 0)),
                  pl.BlockSpec((1, 1, TH, W), lambda b, t: (b, 2, t, 0))],
        out_specs=pl.BlockSpec((1, 1, 5 * M, 1), lambda b, t: (b, t, 0, 0)),
        compiler_params=pltpu.CompilerParams(
            dimension_semantics=("parallel", "parallel"),
            vmem_limit_bytes=_VMEM_LIMIT),
    )(masks, features)

    sums = jnp.sum(partials[..., 0], axis=1).reshape(B, 5, M)
    msum = sums[:, 0]
    denom = msum + 1e-6
    crow = sums[:, 1] / denom
    ccol = sums[:, 2] / denom
    ssum = sums[:, 3]
    s2sum = sums[:, 4]
    s = ssum / denom
    var_loss = jnp.sum(s2sum - 2.0 * s * ssum + HW * s * s) / (B * M * HW)
    s_w = jnp.exp(10.0 * s)
    params = jnp.stack([crow, ccol, s_w], axis=-1)[:, :, None, :]

    keys, seed_parts = pl.pallas_call(
        functools.partial(_emitb_kernel, th=TH, w=W, inv_r=inv_r, inv_c=inv_c),
        out_shape=(jax.ShapeDtypeStruct((B, M, H, W), jnp.uint16),
                   jax.ShapeDtypeStruct((B, nT, M, 1), jnp.float32)),
        grid=(B, nT),
        in_specs=[pl.BlockSpec((1, M, TH, W), lambda b, t: (b, 0, t, 0)),
                  pl.BlockSpec((1, 2, TH, W), lambda b, t: (b, 0, t, 0)),
                  pl.BlockSpec((1, 1, TH, W), lambda b, t: (b, 3, t, 0)),
                  pl.BlockSpec((1, M, 1, 3), lambda b, t: (b, 0, 0, 0))],
        out_specs=[pl.BlockSpec((1, M, TH, W), lambda b, t: (b, 0, t, 0)),
                   pl.BlockSpec((1, 1, M, 1), lambda b, t: (b, t, 0, 0))],
        compiler_params=pltpu.CompilerParams(
            dimension_semantics=("parallel", "parallel"),
            vmem_limit_bytes=_VMEM_LIMIT),
    )(masks, features, features, params)

    seed_loss = jnp.sum(seed_parts) / (B * M * HW)

    flat = keys.reshape(B * M, HW)
    sk = jax.lax.sort(flat, dimension=1, is_stable=False)
    sk3 = sk.reshape(B * M, HW // 128, 128)
    losses = pl.pallas_call(
        functools.partial(_tail_kernel, n=float(HW), qinv=1.0 / _QSCALE),
        out_shape=jax.ShapeDtypeStruct((B * M, 1, 1), jnp.float32),
        grid=(B * M,),
        in_specs=[pl.BlockSpec((1, HW // 128, 128), lambda r: (r, 0, 0))],
        out_specs=pl.BlockSpec((1, 1, 1), lambda r: (r, 0, 0)),
        compiler_params=pltpu.CompilerParams(
            dimension_semantics=("parallel",),
            vmem_limit_bytes=_VMEM_LIMIT),
    )(sk3)
    inst_loss = jnp.mean(losses)

    return w_inst * inst_loss + w_var * var_loss + w_seed * seed_loss
